# baseline (speedup 1.0000x reference)
import functools

import jax
import jax.numpy as jnp
from jax import lax
from jax.experimental import pallas as pl
from jax.experimental.pallas import tpu as pltpu
from jax.experimental.pallas import tpu_sc as plsc

N = 10000
E = 320000
D = 128
G = 64
WROW = 144
NPAD = 10016
NTILES = 32
CH = 112
NCHUNK = 92
EPT = NCHUNK * CH
EPAD = EPT * NTILES
NDP = (N + 16) // 2
BN = 1000
NBLK = N // BN


def _edge_body(hpad_hbm, ad2_hbm, pk_hbm, out_hbm,
               ad2_t, rg0, rg1,
               pk0, pk1, sc0, sc1, dg0, dg1, ds0, ds1, w_v,
               acc_sh, gs0, gs1, ss0, ss1):
    c = lax.axis_index("c")
    s = lax.axis_index("s")
    wid = s * 2 + c
    base_e = wid * EPT

    pltpu.sync_copy(ad2_hbm, ad2_t)

    for b in range(16):
        for j in range(WROW // 16):
            rg0[b, pl.ds(j * 16, 16)] = jnp.zeros((16,), jnp.float32)

    def zstep(k, _):
        pltpu.sync_copy(rg0.at[pl.ds(0, 16)],
                        acc_sh.at[pl.ds(s * 640 + k * 16, 16)])
        return 0
    lax.fori_loop(0, jnp.where(s == 15, 26, 40), zstep, 0)
    plsc.subcore_barrier()

    bufs = ((rg0, pk0, sc0, dg0, ds0, gs0, ss0),
            (rg1, pk1, sc1, dg1, ds1, gs1, ss1))

    def stage_idx(pkb, scur, dgc):
        for j in range(CH // 16):
            pk = pkb[pl.ds(j * 16, 16)]
            scur[pl.ds(j * 16, 16)] = jnp.bitwise_and(pk, 16383)
            dgc[pl.ds(j * 16, 16)] = jnp.right_shift(pk, 14)

    def pk_off(i):
        return base_e + jnp.minimum(i, NCHUNK - 1) * CH

    def compute_w(rg, dgc):
        for j in range(CH // 16):
            rows16 = jnp.arange(16, dtype=jnp.int32) + j * 16
            asv = plsc.load_gather(rg, [rows16,
                                        jnp.full((16,), 129, jnp.int32)])
            dv = dgc[pl.ds(j * 16, 16)]
            pk32 = plsc.load_gather(ad2_t, [jnp.right_shift(dv, 1)])
            half = jnp.right_shift(pk32,
                                   jnp.bitwise_and(dv, 1) * 16)
            adv = plsc.bitcast(jnp.left_shift(half, 16), jnp.float32)
            e = asv + adv
            e = jnp.maximum(e, e * 0.2)
            w_v[pl.ds(j * 16, 16)] = jnp.exp(e)

    def scale_snap(rg, dgc, dsc):
        def grp(gi, _):
            wv = w_v[pl.ds(gi * 16, 16)]
            for l in range(16):
                wl = wv[l]
                b = gi * 16 + l
                for j in range(WROW // 16):
                    rg[b, pl.ds(j * 16, 16)] = rg[b, pl.ds(j * 16, 16)] * wl
            return 0
        lax.fori_loop(0, CH // 16, grp, 0)
        for j in range(CH // 16):
            dsc[pl.ds(j * 16, 16)] = dgc[pl.ds(j * 16, 16)]

    for p in range(2):
        rg, pkb, scur, dgc, dsc, gsem, ssem = bufs[p]
        pltpu.sync_copy(pk_hbm.at[pl.ds(base_e + p * CH, CH)], pkb)
        stage_idx(pkb, scur, dgc)
        pltpu.async_copy(pk_hbm.at[pl.ds(pk_off(2 + p), CH)], pkb, gsem)
    rg, pkb, scur, dgc, dsc, gsem, ssem = bufs[0]
    pltpu.async_copy(hpad_hbm.at[scur], rg, gsem)

    def iteration(i, p, first):
        rg, pkb, scur, dgc, dsc, gsem, ssem = bufs[p]
        rq, pkq, scq, dgq, dsq, gsemq, ssemq = bufs[1 - p]
        pltpu.make_async_copy(hpad_hbm.at[scur], rg, gsem).wait()
        pltpu.make_async_copy(
            pk_hbm.at[pl.ds(pk_off(i + 2), CH)], pkb, gsem).wait()
        compute_w(rg, dgc)
        scale_snap(rg, dgc, dsc)
        pltpu.async_copy(rg, acc_sh.at[dsc], ssem, add=True)
        stage_idx(pkb, scur, dgc)
        pltpu.async_copy(pk_hbm.at[pl.ds(pk_off(i + 4), CH)], pkb, gsem)
        if not first:
            pltpu.make_async_copy(rq, acc_sh.at[dsq], ssemq).wait()
        pltpu.async_copy(hpad_hbm.at[scq], rq, gsemq)

    iteration(0, 0, True)

    def steady(g, _):
        i = 2 * g + 1
        iteration(i, 1, False)
        iteration(i + 1, 0, False)
        return 0
    lax.fori_loop(0, (NCHUNK - 2) // 2, steady, 0)

    iteration(NCHUNK - 1, 1, False)

    rg, pkb, scur, dgc, dsc, gsem, ssem = bufs[1]
    pltpu.make_async_copy(rg, acc_sh.at[dsc], ssem).wait()
    rg, pkb, scur, dgc, dsc, gsem, ssem = bufs[0]
    pltpu.make_async_copy(hpad_hbm.at[scur], rg, gsem).wait()
    pltpu.make_async_copy(
        pk_hbm.at[pl.ds(pk_off(NCHUNK), CH)], pkb, gsem).wait()
    rg, pkb, scur, dgc, dsc, gsem, ssem = bufs[1]
    pltpu.make_async_copy(
        pk_hbm.at[pl.ds(pk_off(NCHUNK), CH)], pkb, gsem).wait()
    plsc.subcore_barrier()

    r0 = s * 624
    pltpu.sync_copy(acc_sh.at[pl.ds(r0, 640)], out_hbm.at[c, pl.ds(r0, 640)])


_edge_pass = functools.partial(
    pl.kernel,
    out_type=jax.ShapeDtypeStruct((2, N, WROW), jnp.float32),
    mesh=plsc.VectorSubcoreMesh(core_axis_name="c", subcore_axis_name="s"),
    compiler_params=pltpu.CompilerParams(
        needs_layout_passes=False, use_tc_tiling_on_sc=False),
    scratch_types=[
        pltpu.VMEM((NDP,), jnp.int32),
        pltpu.VMEM((CH, WROW), jnp.float32),
        pltpu.VMEM((CH, WROW), jnp.float32),
        pltpu.VMEM((CH,), jnp.int32),
        pltpu.VMEM((CH,), jnp.int32),
        pltpu.VMEM((CH,), jnp.int32),
        pltpu.VMEM((CH,), jnp.int32),
        pltpu.VMEM((CH,), jnp.int32),
        pltpu.VMEM((CH,), jnp.int32),
        pltpu.VMEM((CH,), jnp.int32),
        pltpu.VMEM((CH,), jnp.int32),
        pltpu.VMEM((CH,), jnp.float32),
        pltpu.VMEM_SHARED((NPAD, WROW), jnp.float32),
        pltpu.SemaphoreType.DMA,
        pltpu.SemaphoreType.DMA,
        pltpu.SemaphoreType.DMA,
        pltpu.SemaphoreType.DMA,
    ],
)(_edge_body)


def _emit_layer_outputs(h, as_v, ad_v, hpad_ref, as_ref, ad_ref):
    hpad_ref[:, :D] = h
    hpad_ref[:, D:D + 1] = jnp.ones((BN, 1), jnp.float32)
    hpad_ref[:, D + 1:D + 2] = as_v
    hpad_ref[:, D + 2:] = jnp.zeros((BN, WROW - D - 2), jnp.float32)
    as_ref[...] = as_v
    ad_ref[...] = ad_v


def _tc_a_body(x_ref, w_ref, avs_ref, avd_ref, hpad_ref, as_ref, ad_ref):
    h = jnp.dot(x_ref[...], w_ref[...], preferred_element_type=jnp.float32)
    as_v = jnp.dot(h, avs_ref[...], preferred_element_type=jnp.float32)
    ad_v = jnp.dot(h, avd_ref[...], preferred_element_type=jnp.float32)
    _emit_layer_outputs(h, as_v, ad_v, hpad_ref, as_ref, ad_ref)


def _tc_a(x, w, avs, avd):
    return pl.pallas_call(
        _tc_a_body,
        grid=(NBLK,),
        in_specs=[
            pl.BlockSpec((BN, D), lambda i: (i, 0)),
            pl.BlockSpec((D, D), lambda i: (0, 0)),
            pl.BlockSpec((D, 1), lambda i: (0, 0)),
            pl.BlockSpec((D, 1), lambda i: (0, 0)),
        ],
        out_specs=[
            pl.BlockSpec((BN, WROW), lambda i: (i, 0)),
            pl.BlockSpec((BN, 1), lambda i: (i, 0)),
            pl.BlockSpec((BN, 1), lambda i: (i, 0)),
        ],
        out_shape=[
            jax.ShapeDtypeStruct((N, WROW), jnp.float32),
            jax.ShapeDtypeStruct((N, 1), jnp.float32),
            jax.ShapeDtypeStruct((N, 1), jnp.float32),
        ],
    )(x, w, avs, avd)


def _combine(part_ref, b_ref):
    p0 = part_ref[0]
    p1 = part_ref[1]
    den = p0[:, D:D + 1] + p1[:, D:D + 1] + 1e-16
    return (p0[:, :D] + p1[:, :D]) / den + b_ref[...]


def _tc_b_body(part_ref, b_ref, w_ref, avs_ref, avd_ref,
               hpad_ref, as_ref, ad_ref):
    feats = _combine(part_ref, b_ref)
    h = jnp.dot(feats, w_ref[...], preferred_element_type=jnp.float32)
    as_v = jnp.dot(h, avs_ref[...], preferred_element_type=jnp.float32)
    ad_v = jnp.dot(h, avd_ref[...], preferred_element_type=jnp.float32)
    _emit_layer_outputs(h, as_v, ad_v, hpad_ref, as_ref, ad_ref)


def _tc_b(part, b, w, avs, avd):
    return pl.pallas_call(
        _tc_b_body,
        grid=(NBLK,),
        in_specs=[
            pl.BlockSpec((2, BN, WROW), lambda i: (0, i, 0)),
            pl.BlockSpec((1, D), lambda i: (0, 0)),
            pl.BlockSpec((D, D), lambda i: (0, 0)),
            pl.BlockSpec((D, 1), lambda i: (0, 0)),
            pl.BlockSpec((D, 1), lambda i: (0, 0)),
        ],
        out_specs=[
            pl.BlockSpec((BN, WROW), lambda i: (i, 0)),
            pl.BlockSpec((BN, 1), lambda i: (i, 0)),
            pl.BlockSpec((BN, 1), lambda i: (i, 0)),
        ],
        out_shape=[
            jax.ShapeDtypeStruct((N, WROW), jnp.float32),
            jax.ShapeDtypeStruct((N, 1), jnp.float32),
            jax.ShapeDtypeStruct((N, 1), jnp.float32),
        ],
    )(part, b, w, avs, avd)


def _tc_c_body(part_ref, b_ref, batch_ref, out_ref, sums, cnt):
    i = pl.program_id(0)

    @pl.when(i == 0)
    def _():
        sums[...] = jnp.zeros_like(sums)
        cnt[...] = jnp.zeros_like(cnt)

    feats = _combine(part_ref, b_ref)
    bblk = batch_ref[0, 0, :]
    oh = (bblk[None, :] == lax.broadcasted_iota(jnp.int32, (G, BN), 0))
    oh = oh.astype(jnp.float32)
    sums[...] += jnp.dot(oh, feats, preferred_element_type=jnp.float32)
    cnt[...] += jnp.sum(oh, axis=1, keepdims=True)

    @pl.when(i == NBLK - 1)
    def _():
        out_ref[...] = sums[...] / jnp.maximum(cnt[...], 1.0)


def _tc_c(part, b, batch3):
    return pl.pallas_call(
        _tc_c_body,
        grid=(NBLK,),
        in_specs=[
            pl.BlockSpec((2, BN, WROW), lambda i: (0, i, 0)),
            pl.BlockSpec((1, D), lambda i: (0, 0)),
            pl.BlockSpec((1, 1, BN), lambda i: (i, 0, 0)),
        ],
        out_specs=pl.BlockSpec((G, D), lambda i: (0, 0)),
        out_shape=jax.ShapeDtypeStruct((G, D), jnp.float32),
        scratch_shapes=[
            pltpu.VMEM((G, D), jnp.float32),
            pltpu.VMEM((G, 1), jnp.float32),
        ],
    )(part, b, batch3)


def _pack_ad(ad):
    adp = jnp.concatenate([ad.reshape(N), jnp.zeros((16,), jnp.float32)])
    return lax.bitcast_convert_type(
        adp.astype(jnp.bfloat16).reshape(NDP, 2), jnp.int32)


def kernel(x, edge_index, batch, W1, att_src1, att_dst1, b1,
           W2, att_src2, att_dst2, b2):
    srcp = jnp.concatenate(
        [edge_index[0], jnp.zeros((EPAD - E,), jnp.int32)])
    dstp = jnp.concatenate(
        [edge_index[1], jnp.full((EPAD - E,), N, jnp.int32)])
    packed = jnp.bitwise_or(srcp, jnp.left_shift(dstp, 14))

    hpad1, as1, ad1 = _tc_a(x, W1, att_src1.reshape(D, 1),
                            att_dst1.reshape(D, 1))
    part1 = _edge_pass(hpad1, _pack_ad(ad1), packed)
    hpad2, as2, ad2 = _tc_b(part1, b1.reshape(1, D), W2,
                            att_src2.reshape(D, 1), att_dst2.reshape(D, 1))
    part2 = _edge_pass(hpad2, _pack_ad(ad2), packed)
    return _tc_c(part2, b2.reshape(1, D), batch.reshape(NBLK, 1, BN))


# speedup vs baseline: 1.2028x; 1.2028x over previous
import functools

import jax
import jax.numpy as jnp
from jax import lax
from jax.experimental import pallas as pl
from jax.experimental.pallas import tpu as pltpu
from jax.experimental.pallas import tpu_sc as plsc

N = 10000
E = 320000
D = 128
G = 64
WROW = 144
NPAD = 10000
NTILES = 32
CH = 128
NCHUNK = 80
EPT = NCHUNK * CH
EPAD = EPT * NTILES
BN = 1000
NBLK = N // BN


def _edge_body(hpad_hbm, ad_hbm, pk_hbm, out_hbm,
               rg0, rg1, adv0, adv1,
               pk0, pk1, sc0, sc1, dg0, dg1, w_v,
               acc_sh, gs0, gs1, ps0, ps1):
    c = lax.axis_index("c")
    s = lax.axis_index("s")
    wid = s * 2 + c
    base_e = wid * EPT

    for b in range(16):
        for j in range(WROW // 16):
            rg0[b, pl.ds(j * 16, 16)] = jnp.zeros((16,), jnp.float32)

    def zstep(k, _):
        pltpu.sync_copy(rg0.at[pl.ds(0, 16)],
                        acc_sh.at[pl.ds(s * 625 + k * 16, 16)])
        return 0
    lax.fori_loop(0, 39, zstep, 0)
    pltpu.sync_copy(rg0.at[pl.ds(0, 1)], acc_sh.at[pl.ds(s * 625 + 624, 1)])
    plsc.subcore_barrier()

    bufs = ((rg0, adv0, pk0, sc0, dg0, gs0, ps0),
            (rg1, adv1, pk1, sc1, dg1, gs1, ps1))

    def stage_idx(pkb, scur, dgc):
        for j in range(CH // 16):
            pk = pkb[pl.ds(j * 16, 16)]
            scur[pl.ds(j * 16, 16)] = jnp.bitwise_and(pk, 16383)
            dgc[pl.ds(j * 16, 16)] = jnp.right_shift(pk, 14)

    def pk_off(i):
        return base_e + jnp.minimum(i, NCHUNK - 1) * CH

    def compute_w(rg, adv):
        for j in range(CH // 16):
            rows16 = jnp.arange(16, dtype=jnp.int32) + j * 16
            asv = plsc.load_gather(rg, [rows16,
                                        jnp.full((16,), 129, jnp.int32)])
            e = asv + adv[pl.ds(j * 16, 16)]
            e = jnp.maximum(e, e * 0.2)
            w_v[pl.ds(j * 16, 16)] = jnp.exp(e)

    def scale(rg):
        def grp(gi, _):
            wv = w_v[pl.ds(gi * 16, 16)]
            for l in range(16):
                wl = wv[l]
                b = gi * 16 + l
                for j in range(WROW // 16):
                    rg[b, pl.ds(j * 16, 16)] = rg[b, pl.ds(j * 16, 16)] * wl
            return 0
        lax.fori_loop(0, CH // 16, grp, 0)

    for p in range(2):
        rg, adv, pkb, scur, dgc, gsem, psem = bufs[p]
        pltpu.sync_copy(pk_hbm.at[pl.ds(base_e + p * CH, CH)], pkb)
        stage_idx(pkb, scur, dgc)
        pltpu.async_copy(pk_hbm.at[pl.ds(pk_off(2 + p), CH)], pkb, psem)
        pltpu.async_copy(hpad_hbm.at[scur], rg, gsem)
        pltpu.async_copy(ad_hbm.at[dgc], adv, gsem)

    def iteration(i, p):
        rg, adv, pkb, scur, dgc, gsem, psem = bufs[p]
        pltpu.make_async_copy(hpad_hbm.at[scur], rg, gsem).wait()
        pltpu.make_async_copy(ad_hbm.at[dgc], adv, gsem).wait()
        compute_w(rg, adv)
        scale(rg)
        pltpu.sync_copy(rg, acc_sh.at[dgc], add=True)
        pltpu.make_async_copy(
            pk_hbm.at[pl.ds(pk_off(i + 2), CH)], pkb, psem).wait()
        stage_idx(pkb, scur, dgc)
        pltpu.async_copy(pk_hbm.at[pl.ds(pk_off(i + 4), CH)], pkb, psem)
        pltpu.async_copy(hpad_hbm.at[scur], rg, gsem)
        pltpu.async_copy(ad_hbm.at[dgc], adv, gsem)

    def steady(g, _):
        iteration(2 * g, 0)
        iteration(2 * g + 1, 1)
        return 0
    lax.fori_loop(0, NCHUNK // 2, steady, 0)

    for p in range(2):
        rg, adv, pkb, scur, dgc, gsem, psem = bufs[p]
        pltpu.make_async_copy(hpad_hbm.at[scur], rg, gsem).wait()
        pltpu.make_async_copy(ad_hbm.at[dgc], adv, gsem).wait()
        pltpu.make_async_copy(
            pk_hbm.at[pl.ds(pk_off(NCHUNK), CH)], pkb, psem).wait()
    plsc.subcore_barrier()

    r0 = s * 625
    pltpu.sync_copy(acc_sh.at[pl.ds(r0, 625)], out_hbm.at[c, pl.ds(r0, 625)])


_edge_pass = functools.partial(
    pl.kernel,
    out_type=jax.ShapeDtypeStruct((2, N, WROW), jnp.float32),
    mesh=plsc.VectorSubcoreMesh(core_axis_name="c", subcore_axis_name="s"),
    compiler_params=pltpu.CompilerParams(
        needs_layout_passes=False, use_tc_tiling_on_sc=False),
    scratch_types=[
        pltpu.VMEM((CH, WROW), jnp.float32),
        pltpu.VMEM((CH, WROW), jnp.float32),
        pltpu.VMEM((CH,), jnp.float32),
        pltpu.VMEM((CH,), jnp.float32),
        pltpu.VMEM((CH,), jnp.int32),
        pltpu.VMEM((CH,), jnp.int32),
        pltpu.VMEM((CH,), jnp.int32),
        pltpu.VMEM((CH,), jnp.int32),
        pltpu.VMEM((CH,), jnp.int32),
        pltpu.VMEM((CH,), jnp.int32),
        pltpu.VMEM((CH,), jnp.float32),
        pltpu.VMEM_SHARED((NPAD, WROW), jnp.float32),
        pltpu.SemaphoreType.DMA,
        pltpu.SemaphoreType.DMA,
        pltpu.SemaphoreType.DMA,
        pltpu.SemaphoreType.DMA,
    ],
)(_edge_body)


def _emit_layer_outputs(h, as_v, ad_ref, ad_v, hpad_ref):
    hpad_ref[:, :D] = h
    hpad_ref[:, D:D + 1] = jnp.ones((BN, 1), jnp.float32)
    hpad_ref[:, D + 1:D + 2] = as_v
    hpad_ref[:, D + 2:] = jnp.zeros((BN, WROW - D - 2), jnp.float32)
    ad_ref[...] = ad_v


def _tc_a_body(x_ref, w_ref, avs_ref, avd_ref, hpad_ref, ad_ref):
    h = jnp.dot(x_ref[...], w_ref[...], preferred_element_type=jnp.float32)
    as_v = jnp.dot(h, avs_ref[...], preferred_element_type=jnp.float32)
    ad_v = jnp.dot(h, avd_ref[...], preferred_element_type=jnp.float32)
    _emit_layer_outputs(h, as_v, ad_ref, ad_v, hpad_ref)


def _tc_a(x, w, avs, avd):
    return pl.pallas_call(
        _tc_a_body,
        grid=(NBLK,),
        in_specs=[
            pl.BlockSpec((BN, D), lambda i: (i, 0)),
            pl.BlockSpec((D, D), lambda i: (0, 0)),
            pl.BlockSpec((D, 1), lambda i: (0, 0)),
            pl.BlockSpec((D, 1), lambda i: (0, 0)),
        ],
        out_specs=[
            pl.BlockSpec((BN, WROW), lambda i: (i, 0)),
            pl.BlockSpec((BN, 1), lambda i: (i, 0)),
        ],
        out_shape=[
            jax.ShapeDtypeStruct((N, WROW), jnp.float32),
            jax.ShapeDtypeStruct((N, 1), jnp.float32),
        ],
    )(x, w, avs, avd)


def _combine(part_ref, b_ref):
    p0 = part_ref[0]
    p1 = part_ref[1]
    den = p0[:, D:D + 1] + p1[:, D:D + 1] + 1e-16
    return (p0[:, :D] + p1[:, :D]) / den + b_ref[...]


def _tc_b_body(part_ref, b_ref, w_ref, avs_ref, avd_ref, hpad_ref, ad_ref):
    feats = _combine(part_ref, b_ref)
    h = jnp.dot(feats, w_ref[...], preferred_element_type=jnp.float32)
    as_v = jnp.dot(h, avs_ref[...], preferred_element_type=jnp.float32)
    ad_v = jnp.dot(h, avd_ref[...], preferred_element_type=jnp.float32)
    _emit_layer_outputs(h, as_v, ad_ref, ad_v, hpad_ref)


def _tc_b(part, b, w, avs, avd):
    return pl.pallas_call(
        _tc_b_body,
        grid=(NBLK,),
        in_specs=[
            pl.BlockSpec((2, BN, WROW), lambda i: (0, i, 0)),
            pl.BlockSpec((1, D), lambda i: (0, 0)),
            pl.BlockSpec((D, D), lambda i: (0, 0)),
            pl.BlockSpec((D, 1), lambda i: (0, 0)),
            pl.BlockSpec((D, 1), lambda i: (0, 0)),
        ],
        out_specs=[
            pl.BlockSpec((BN, WROW), lambda i: (i, 0)),
            pl.BlockSpec((BN, 1), lambda i: (i, 0)),
        ],
        out_shape=[
            jax.ShapeDtypeStruct((N, WROW), jnp.float32),
            jax.ShapeDtypeStruct((N, 1), jnp.float32),
        ],
    )(part, b, w, avs, avd)


def _tc_c_body(part_ref, b_ref, batch_ref, out_ref, sums, cnt):
    i = pl.program_id(0)

    @pl.when(i == 0)
    def _():
        sums[...] = jnp.zeros_like(sums)
        cnt[...] = jnp.zeros_like(cnt)

    feats = _combine(part_ref, b_ref)
    bblk = batch_ref[0, 0, :]
    oh = (bblk[None, :] == lax.broadcasted_iota(jnp.int32, (G, BN), 0))
    oh = oh.astype(jnp.float32)
    sums[...] += jnp.dot(oh, feats, preferred_element_type=jnp.float32)
    cnt[...] += jnp.sum(oh, axis=1, keepdims=True)

    @pl.when(i == NBLK - 1)
    def _():
        out_ref[...] = sums[...] / jnp.maximum(cnt[...], 1.0)


def _tc_c(part, b, batch3):
    return pl.pallas_call(
        _tc_c_body,
        grid=(NBLK,),
        in_specs=[
            pl.BlockSpec((2, BN, WROW), lambda i: (0, i, 0)),
            pl.BlockSpec((1, D), lambda i: (0, 0)),
            pl.BlockSpec((1, 1, BN), lambda i: (i, 0, 0)),
        ],
        out_specs=pl.BlockSpec((G, D), lambda i: (0, 0)),
        out_shape=jax.ShapeDtypeStruct((G, D), jnp.float32),
        scratch_shapes=[
            pltpu.VMEM((G, D), jnp.float32),
            pltpu.VMEM((G, 1), jnp.float32),
        ],
    )(part, b, batch3)


def kernel(x, edge_index, batch, W1, att_src1, att_dst1, b1,
           W2, att_src2, att_dst2, b2):
    srcp = jnp.concatenate(
        [edge_index[0], jnp.full((EPAD - E,), N, jnp.int32)])
    dstp = jnp.concatenate(
        [edge_index[1], jnp.zeros((EPAD - E,), jnp.int32)])
    packed = jnp.bitwise_or(srcp, jnp.left_shift(dstp, 14))
    zrow = jnp.zeros((8, WROW), jnp.float32)

    hpad1, ad1 = _tc_a(x, W1, att_src1.reshape(D, 1), att_dst1.reshape(D, 1))
    part1 = _edge_pass(jnp.concatenate([hpad1, zrow]), ad1.reshape(N), packed)
    hpad2, ad2 = _tc_b(part1, b1.reshape(1, D), W2,
                       att_src2.reshape(D, 1), att_dst2.reshape(D, 1))
    part2 = _edge_pass(jnp.concatenate([hpad2, zrow]), ad2.reshape(N), packed)
    return _tc_c(part2, b2.reshape(1, D), batch.reshape(NBLK, 1, BN))


# speedup vs baseline: 1.2712x; 1.0569x over previous
import functools

import jax
import jax.numpy as jnp
from jax import lax
from jax.experimental import pallas as pl
from jax.experimental.pallas import tpu as pltpu
from jax.experimental.pallas import tpu_sc as plsc

N = 10000
E = 320000
D = 128
G = 64
WROW = 144
NPAD = 10000
NTILES = 32
CH = 128
NCHUNK = 80
EPT = NCHUNK * CH
EPAD = EPT * NTILES
BN = 1000
NBLK = N // BN


def _edge_body(hpad_hbm, ad_hbm, pk_hbm, out_hbm,
               rg0, rg1, adv0, adv1,
               pk0, pk1, sc0, sc1, dg0, dg1, w_v,
               acc_sh, gs0, gs1, ps0, ps1):
    c = lax.axis_index("c")
    s = lax.axis_index("s")
    wid = s * 2 + c
    base_e = wid * EPT

    for b in range(16):
        for j in range(WROW // 16):
            rg0[b, pl.ds(j * 16, 16)] = jnp.zeros((16,), jnp.float32)

    def zstep(k, _):
        pltpu.sync_copy(rg0.at[pl.ds(0, 16)],
                        acc_sh.at[pl.ds(s * 625 + k * 16, 16)])
        return 0
    lax.fori_loop(0, 39, zstep, 0)
    pltpu.sync_copy(rg0.at[pl.ds(0, 1)], acc_sh.at[pl.ds(s * 625 + 624, 1)])
    plsc.subcore_barrier()

    bufs = ((rg0, adv0, pk0, sc0, dg0, gs0, ps0),
            (rg1, adv1, pk1, sc1, dg1, gs1, ps1))

    def stage_idx(pkb, scur, dgc):
        for j in range(CH // 16):
            pk = pkb[pl.ds(j * 16, 16)]
            scur[pl.ds(j * 16, 16)] = jnp.bitwise_and(pk, 16383)
            dgc[pl.ds(j * 16, 16)] = jnp.right_shift(pk, 14)

    def pk_off(i):
        return base_e + jnp.minimum(i, NCHUNK - 1) * CH

    def compute_w(rg, adv):
        for j in range(CH // 16):
            rows16 = jnp.arange(16, dtype=jnp.int32) + j * 16
            asv = plsc.load_gather(rg, [rows16,
                                        jnp.full((16,), 129, jnp.int32)])
            e = asv + adv[pl.ds(j * 16, 16)]
            e = jnp.maximum(e, e * 0.2)
            w_v[pl.ds(j * 16, 16)] = jnp.exp(e)

    def scale(rg):
        def grp(gi, _):
            wv = w_v[pl.ds(gi * 16, 16)]
            for l in range(16):
                wl = wv[l]
                b = gi * 16 + l
                for j in range(WROW // 16):
                    rg[b, pl.ds(j * 16, 16)] = rg[b, pl.ds(j * 16, 16)] * wl
            return 0
        lax.fori_loop(0, CH // 16, grp, 0)

    for p in range(2):
        rg, adv, pkb, scur, dgc, gsem, psem = bufs[p]
        pltpu.sync_copy(pk_hbm.at[pl.ds(base_e + p * CH, CH)], pkb)
        stage_idx(pkb, scur, dgc)
        pltpu.async_copy(pk_hbm.at[pl.ds(pk_off(2 + p), CH)], pkb, psem)
        pltpu.async_copy(hpad_hbm.at[scur], rg, gsem)
        pltpu.async_copy(ad_hbm.at[dgc], adv, gsem)

    def iteration(i, p):
        rg, adv, pkb, scur, dgc, gsem, psem = bufs[p]
        pltpu.make_async_copy(hpad_hbm.at[scur], rg, gsem).wait()
        pltpu.make_async_copy(ad_hbm.at[dgc], adv, gsem).wait()
        compute_w(rg, adv)
        scale(rg)
        pltpu.sync_copy(rg, acc_sh.at[dgc], add=True)
        pltpu.make_async_copy(
            pk_hbm.at[pl.ds(pk_off(i + 2), CH)], pkb, psem).wait()
        stage_idx(pkb, scur, dgc)
        pltpu.async_copy(pk_hbm.at[pl.ds(pk_off(i + 4), CH)], pkb, psem)
        pltpu.async_copy(hpad_hbm.at[scur], rg, gsem)
        pltpu.async_copy(ad_hbm.at[dgc], adv, gsem)

    def steady(g, _):
        iteration(2 * g, 0)
        iteration(2 * g + 1, 1)
        return 0
    lax.fori_loop(0, NCHUNK // 2, steady, 0)

    for p in range(2):
        rg, adv, pkb, scur, dgc, gsem, psem = bufs[p]
        pltpu.make_async_copy(hpad_hbm.at[scur], rg, gsem).wait()
        pltpu.make_async_copy(ad_hbm.at[dgc], adv, gsem).wait()
        pltpu.make_async_copy(
            pk_hbm.at[pl.ds(pk_off(NCHUNK), CH)], pkb, psem).wait()
    plsc.subcore_barrier()

    r0 = s * 625
    pltpu.sync_copy(acc_sh.at[pl.ds(r0, 625)], out_hbm.at[c, pl.ds(r0, 625)])


_edge_pass = functools.partial(
    pl.kernel,
    out_type=jax.ShapeDtypeStruct((2, N, WROW), jnp.float32),
    mesh=plsc.VectorSubcoreMesh(core_axis_name="c", subcore_axis_name="s"),
    compiler_params=pltpu.CompilerParams(
        needs_layout_passes=False, use_tc_tiling_on_sc=False),
    scratch_types=[
        pltpu.VMEM((CH, WROW), jnp.float32),
        pltpu.VMEM((CH, WROW), jnp.float32),
        pltpu.VMEM((CH,), jnp.float32),
        pltpu.VMEM((CH,), jnp.float32),
        pltpu.VMEM((CH,), jnp.int32),
        pltpu.VMEM((CH,), jnp.int32),
        pltpu.VMEM((CH,), jnp.int32),
        pltpu.VMEM((CH,), jnp.int32),
        pltpu.VMEM((CH,), jnp.int32),
        pltpu.VMEM((CH,), jnp.int32),
        pltpu.VMEM((CH,), jnp.float32),
        pltpu.VMEM_SHARED((NPAD, WROW), jnp.float32),
        pltpu.SemaphoreType.DMA,
        pltpu.SemaphoreType.DMA,
        pltpu.SemaphoreType.DMA,
        pltpu.SemaphoreType.DMA,
    ],
)(_edge_body)


def _emit_layer_outputs(h, as_v, ad_ref, ad_v, hpad_ref):
    hpad_ref[:, :D] = h
    hpad_ref[:, D:D + 1] = jnp.ones((BN, 1), jnp.float32)
    hpad_ref[:, D + 1:D + 2] = as_v
    hpad_ref[:, D + 2:] = jnp.zeros((BN, WROW - D - 2), jnp.float32)
    ad_ref[...] = ad_v


def _tc_a_body(x_ref, w_ref, avs_ref, avd_ref, hpad_ref, ad_ref):
    h = jnp.dot(x_ref[...], w_ref[...], preferred_element_type=jnp.float32)
    as_v = jnp.dot(h, avs_ref[...], preferred_element_type=jnp.float32)
    ad_v = jnp.dot(h, avd_ref[...], preferred_element_type=jnp.float32)
    _emit_layer_outputs(h, as_v, ad_ref, ad_v, hpad_ref)


def _tc_a(x, w, avs, avd):
    return pl.pallas_call(
        _tc_a_body,
        grid=(NBLK,),
        in_specs=[
            pl.BlockSpec((BN, D), lambda i: (i, 0)),
            pl.BlockSpec((D, D), lambda i: (0, 0)),
            pl.BlockSpec((D, 1), lambda i: (0, 0)),
            pl.BlockSpec((D, 1), lambda i: (0, 0)),
        ],
        out_specs=[
            pl.BlockSpec((BN, WROW), lambda i: (i, 0)),
            pl.BlockSpec((BN, 1), lambda i: (i, 0)),
        ],
        out_shape=[
            jax.ShapeDtypeStruct((N, WROW), jnp.float32),
            jax.ShapeDtypeStruct((N, 1), jnp.float32),
        ],
    )(x, w, avs, avd)


def _combine(part_ref, b_ref):
    p0 = part_ref[0]
    p1 = part_ref[1]
    den = p0[:, D:D + 1] + p1[:, D:D + 1] + 1e-16
    return (p0[:, :D] + p1[:, :D]) / den + b_ref[...]


def _tc_b_body(part_ref, b_ref, w_ref, avs_ref, avd_ref, hpad_ref, ad_ref):
    feats = _combine(part_ref, b_ref)
    h = jnp.dot(feats, w_ref[...], preferred_element_type=jnp.float32)
    as_v = jnp.dot(h, avs_ref[...], preferred_element_type=jnp.float32)
    ad_v = jnp.dot(h, avd_ref[...], preferred_element_type=jnp.float32)
    _emit_layer_outputs(h, as_v, ad_ref, ad_v, hpad_ref)


def _tc_b(part, b, w, avs, avd):
    return pl.pallas_call(
        _tc_b_body,
        grid=(NBLK,),
        in_specs=[
            pl.BlockSpec((2, BN, WROW), lambda i: (0, i, 0)),
            pl.BlockSpec((1, D), lambda i: (0, 0)),
            pl.BlockSpec((D, D), lambda i: (0, 0)),
            pl.BlockSpec((D, 1), lambda i: (0, 0)),
            pl.BlockSpec((D, 1), lambda i: (0, 0)),
        ],
        out_specs=[
            pl.BlockSpec((BN, WROW), lambda i: (i, 0)),
            pl.BlockSpec((BN, 1), lambda i: (i, 0)),
        ],
        out_shape=[
            jax.ShapeDtypeStruct((N, WROW), jnp.float32),
            jax.ShapeDtypeStruct((N, 1), jnp.float32),
        ],
    )(part, b, w, avs, avd)


def _tc_c_body(part_ref, b_ref, batch_ref, out_ref, sums, cnt):
    i = pl.program_id(0)

    @pl.when(i == 0)
    def _():
        sums[...] = jnp.zeros_like(sums)
        cnt[...] = jnp.zeros_like(cnt)

    feats = _combine(part_ref, b_ref)
    bblk = batch_ref[0, 0, :]
    oh = (bblk[None, :] == lax.broadcasted_iota(jnp.int32, (G, BN), 0))
    oh = oh.astype(jnp.float32)
    sums[...] += jnp.dot(oh, feats, preferred_element_type=jnp.float32)
    cnt[...] += jnp.sum(oh, axis=1, keepdims=True)

    @pl.when(i == NBLK - 1)
    def _():
        out_ref[...] = sums[...] / jnp.maximum(cnt[...], 1.0)


def _tc_c(part, b, batch3):
    return pl.pallas_call(
        _tc_c_body,
        grid=(NBLK,),
        in_specs=[
            pl.BlockSpec((2, BN, WROW), lambda i: (0, i, 0)),
            pl.BlockSpec((1, D), lambda i: (0, 0)),
            pl.BlockSpec((1, 1, BN), lambda i: (i, 0, 0)),
        ],
        out_specs=pl.BlockSpec((G, D), lambda i: (0, 0)),
        out_shape=jax.ShapeDtypeStruct((G, D), jnp.float32),
        scratch_shapes=[
            pltpu.VMEM((G, D), jnp.float32),
            pltpu.VMEM((G, 1), jnp.float32),
        ],
    )(part, b, batch3)


def kernel(x, edge_index, batch, W1, att_src1, att_dst1, b1,
           W2, att_src2, att_dst2, b2):
    srcp = jnp.concatenate(
        [edge_index[0], jnp.full((EPAD - E,), N, jnp.int32)])
    dstp = jnp.concatenate(
        [edge_index[1],
         jnp.arange(EPAD - E, dtype=jnp.int32) * 13 % N])
    packed = jnp.bitwise_or(srcp, jnp.left_shift(dstp, 14))
    zrow = jnp.zeros((8, WROW), jnp.float32)

    hpad1, ad1 = _tc_a(x, W1, att_src1.reshape(D, 1), att_dst1.reshape(D, 1))
    part1 = _edge_pass(jnp.concatenate([hpad1, zrow]), ad1.reshape(N), packed)
    hpad2, ad2 = _tc_b(part1, b1.reshape(1, D), W2,
                       att_src2.reshape(D, 1), att_dst2.reshape(D, 1))
    part2 = _edge_pass(jnp.concatenate([hpad2, zrow]), ad2.reshape(N), packed)
    return _tc_c(part2, b2.reshape(1, D), batch.reshape(NBLK, 1, BN))
